# Initial kernel scaffold; baseline (speedup 1.0000x reference)
import jax
import jax.numpy as jnp
from jax.experimental import pallas as pl


def kernel(pos, t, batch, W1_0, b1_0, W1_1, b1_1, W1_2, b1_2, W2_0, b2_0, W2_1, b2_1, W2_2, b2_2, W0, b0, Wl1, bl1, Wl2, bl2):
    raise NotImplementedError("write your pallas kernel here")


# speedup vs baseline: 8.9592x; 8.9592x over previous
import functools

import jax
import jax.numpy as jnp
from jax import lax
from jax.experimental import pallas as pl
from jax.experimental.pallas import tpu as pltpu
from jax.experimental.pallas import tpu_sc as plsc

B = 16
P = 512
N = B * P
K = 30


def _bdot(a, b):
    return lax.dot_general(a.astype(jnp.bfloat16), b.astype(jnp.bfloat16),
                           (((1,), (0,)), ((), ())),
                           preferred_element_type=jnp.float32)


def _knn_body(x_ref, idx_ref, d_ref):
    boff = pl.program_id(0) * P
    x = x_ref[...]
    xh = x.astype(jnp.bfloat16)
    prod = lax.dot_general(xh, xh, (((1,), (1,)), ((), ())),
                           preferred_element_type=jnp.float32)
    sq = jnp.sum(x * x, axis=1, keepdims=True)
    iota0 = lax.broadcasted_iota(jnp.int32, (P, P), 0)
    iota1 = lax.broadcasted_iota(jnp.int32, (P, P), 1)
    d_ref[...] = jnp.where(iota0 == iota1, jnp.float32(1e10), sq - 2.0 * prod)

    def step(k, _):
        d = d_ref[...]
        colmin = jnp.min(d, axis=0, keepdims=True)
        cand = jnp.where(d == colmin, iota0, P)
        sel = jnp.min(cand, axis=0, keepdims=True)
        idx_ref[0, pl.ds(k, 1), :] = sel + boff
        d_ref[...] = jnp.where(iota0 == sel, jnp.float32(2e10), d)
        return 0

    lax.fori_loop(0, K, step, 0)


def _knn_call(x):
    din = x.shape[1]
    return pl.pallas_call(
        _knn_body,
        grid=(B,),
        in_specs=[pl.BlockSpec((P, din), lambda b: (b, 0))],
        out_specs=pl.BlockSpec((1, K, P), lambda b: (b, 0, 0)),
        out_shape=jax.ShapeDtypeStruct((B, K, P), jnp.int32),
        scratch_shapes=[pltpu.VMEM((P, P), jnp.float32)],
    )(x)


def _sc_gather(table, idx):
    e_total = idx.shape[0]
    d = table.shape[1]
    info = plsc.get_sparse_core_info()
    nw = info.num_cores * info.num_subcores
    e_per_w = e_total // nw
    ch = 128
    n_ch = e_per_w // ch
    mesh = plsc.VectorSubcoreMesh(core_axis_name="c", subcore_axis_name="s")

    @functools.partial(
        pl.kernel,
        mesh=mesh,
        out_type=jax.ShapeDtypeStruct((e_total, d), table.dtype),
        scratch_types=[
            pltpu.VMEM((e_per_w,), jnp.int32),
            pltpu.VMEM((2, ch, d), table.dtype),
            pltpu.SemaphoreType.DMA,
            pltpu.SemaphoreType.DMA,
        ],
    )
    def gather_k(table_hbm, idx_hbm, out_hbm, idx_v, rows_v, sem0, sem1):
        wid = lax.axis_index("s") * info.num_cores + lax.axis_index("c")
        base = wid * e_per_w
        pltpu.sync_copy(idx_hbm.at[pl.ds(base, e_per_w)], idx_v)
        sems = (sem0, sem1)

        def fire(c, buf):
            pltpu.async_copy(
                table_hbm.at[idx_v.at[pl.ds(c * ch, ch)]],
                rows_v.at[buf], sems[buf])

        def drain_and_store(c, buf):
            pltpu.make_async_copy(
                table_hbm.at[idx_v.at[pl.ds(c * ch, ch)]],
                rows_v.at[buf], sems[buf]).wait()
            pltpu.sync_copy(rows_v.at[buf],
                            out_hbm.at[pl.ds(base + c * ch, ch)])

        fire(0, 0)

        def loop_body(c, _):
            @pl.when(c + 1 < n_ch)
            def _fire_next():
                @pl.when(lax.rem(c + 1, 2) == 0)
                def _():
                    fire(c + 1, 0)

                @pl.when(lax.rem(c + 1, 2) == 1)
                def _():
                    fire(c + 1, 1)

            @pl.when(lax.rem(c, 2) == 0)
            def _w0():
                drain_and_store(c, 0)

            @pl.when(lax.rem(c, 2) == 1)
            def _w1():
                drain_and_store(c, 1)
            return 0

        lax.fori_loop(0, n_ch, loop_body, 0)

    return gather_k(table, idx)


def _edge_body(x_ref, g_ref, w0_ref, b0_ref, w1_ref, b1_ref, w2_ref, b2_ref,
               out_ref, *, c, din):
    i = pl.program_id(1)
    f2 = w2_ref.shape[1]
    xi = x_ref[pl.ds(i * c, c), :]
    xj = g_ref[0][:, :, :din]
    xib = jnp.broadcast_to(xi[None, :, :], (K, c, din))
    ein = jnp.concatenate([xib, xj - xib], axis=2).reshape(K * c, 2 * din)
    h = jnp.maximum(_bdot(ein, w0_ref[...]) + b0_ref[...], 0.0)
    h = jnp.maximum(_bdot(h, w1_ref[...]) + b1_ref[...], 0.0)
    h = jnp.maximum(_bdot(h, w2_ref[...]) + b2_ref[...], 0.0)
    out_ref[...] = jnp.max(h.reshape(K, c, f2), axis=0)


def _edge_call(x, g4, w0, b0, w1, b1, w2, b2):
    din = x.shape[1]
    f1 = w0.shape[1]
    f2 = w2.shape[1]
    c = 128
    nc = P // c
    return pl.pallas_call(
        functools.partial(_edge_body, c=c, din=din),
        grid=(B, nc),
        in_specs=[
            pl.BlockSpec((P, din), lambda b, i: (b, 0)),
            pl.BlockSpec((1, K, c, 128), lambda b, i: (b, 0, i, 0)),
            pl.BlockSpec((2 * din, f1), lambda b, i: (0, 0)),
            pl.BlockSpec((1, f1), lambda b, i: (0, 0)),
            pl.BlockSpec((f1, f1), lambda b, i: (0, 0)),
            pl.BlockSpec((1, f1), lambda b, i: (0, 0)),
            pl.BlockSpec((f1, f2), lambda b, i: (0, 0)),
            pl.BlockSpec((1, f2), lambda b, i: (0, 0)),
        ],
        out_specs=pl.BlockSpec((c, f2), lambda b, i: (b * nc + i, 0)),
        out_shape=jax.ShapeDtypeStruct((N, f2), jnp.float32),
    )(x, g4, w0, b0.reshape(1, f1), w1, b1.reshape(1, f1),
      w2, b2.reshape(1, f2))


def _head_body(x1_ref, x2_ref, w0_ref, b0_ref, wl1a_ref, wl1b_ref, wl1c_ref,
               bl1_ref, wl2_ref, bl2_ref, out_ref):
    x1 = x1_ref[...]
    x2 = x2_ref[...]
    h = jnp.maximum(_bdot(x2, w0_ref[...]) + b0_ref[...], 0.0)
    pooled = jnp.max(h, axis=0, keepdims=True)
    z = (_bdot(x1, wl1a_ref[...]) + _bdot(x2, wl1b_ref[...])
         + _bdot(pooled, wl1c_ref[...]) + bl1_ref[...])
    z = jnp.maximum(z, 0.0)
    out_ref[...] = jnp.maximum(_bdot(z, wl2_ref[...]) + bl2_ref[...], 0.0)


def _head_call(x1, x2, w0, b0, wl1, bl1, wl2, bl2):
    return pl.pallas_call(
        _head_body,
        grid=(B,),
        in_specs=[
            pl.BlockSpec((P, 64), lambda b: (b, 0)),
            pl.BlockSpec((P, 256), lambda b: (b, 0)),
            pl.BlockSpec((256, 512), lambda b: (0, 0)),
            pl.BlockSpec((1, 512), lambda b: (0, 0)),
            pl.BlockSpec((64, 256), lambda b: (0, 0)),
            pl.BlockSpec((256, 256), lambda b: (0, 0)),
            pl.BlockSpec((512, 256), lambda b: (0, 0)),
            pl.BlockSpec((1, 256), lambda b: (0, 0)),
            pl.BlockSpec((256, 256), lambda b: (0, 0)),
            pl.BlockSpec((1, 256), lambda b: (0, 0)),
        ],
        out_specs=pl.BlockSpec((P, 256), lambda b: (b, 0)),
        out_shape=jax.ShapeDtypeStruct((N, 256), jnp.float32),
    )(x1, x2, w0, b0.reshape(1, 512), wl1[:64], wl1[64:320], wl1[320:],
      bl1.reshape(1, 256), wl2, bl2.reshape(1, 256))


def _pad128(x):
    return jnp.concatenate(
        [x, jnp.zeros((x.shape[0], 128 - x.shape[1]), x.dtype)], axis=1)


def kernel(pos, t, batch, W1_0, b1_0, W1_1, b1_1, W1_2, b1_2,
           W2_0, b2_0, W2_1, b2_1, W2_2, b2_2, W0, b0, Wl1, bl1, Wl2, bl2):
    del t, batch
    idx1 = _knn_call(pos)
    g1 = _sc_gather(_pad128(pos), idx1.reshape(-1))
    x1 = _edge_call(pos, g1.reshape(B, K, P, 128), W1_0, b1_0,
                    W1_1, b1_1, W1_2, b1_2)

    idx2 = _knn_call(x1)
    g2 = _sc_gather(_pad128(x1), idx2.reshape(-1))
    x2 = _edge_call(x1, g2.reshape(B, K, P, 128), W2_0, b2_0,
                    W2_1, b2_1, W2_2, b2_2)

    return _head_call(x1, x2, W0, b0, Wl1, bl1, Wl2, bl2)


# speedup vs baseline: 9.6101x; 1.0727x over previous
import functools

import jax
import jax.numpy as jnp
from jax import lax
from jax.experimental import pallas as pl
from jax.experimental.pallas import tpu as pltpu
from jax.experimental.pallas import tpu_sc as plsc

B = 16
P = 512
N = B * P
K = 30


def _bdot(a, b):
    return lax.dot_general(a.astype(jnp.bfloat16), b.astype(jnp.bfloat16),
                           (((1,), (0,)), ((), ())),
                           preferred_element_type=jnp.float32)


def _knn_body(x_ref, idx_ref, d_ref):
    boff = pl.program_id(0) * P
    x = x_ref[...]
    xh = x.astype(jnp.bfloat16)
    prod = lax.dot_general(xh, xh, (((1,), (1,)), ((), ())),
                           preferred_element_type=jnp.float32)
    sq = jnp.sum(x * x, axis=1, keepdims=True)
    iota0 = lax.broadcasted_iota(jnp.int32, (P, P), 0)
    iota1 = lax.broadcasted_iota(jnp.int32, (P, P), 1)
    d_ref[...] = jnp.where(iota0 == iota1, jnp.float32(1e10), sq - 2.0 * prod)

    def step(k, _):
        d = d_ref[...]
        colmin = jnp.min(d, axis=0, keepdims=True)
        cand = jnp.where(d == colmin, iota0, P)
        sel = jnp.min(cand, axis=0, keepdims=True)
        idx_ref[0, pl.ds(k, 1), :] = sel + boff
        d_ref[...] = jnp.where(iota0 == sel, jnp.float32(2e10), d)
        return 0

    lax.fori_loop(0, K, step, 0)


def _knn_call(x):
    din = x.shape[1]
    nb = x.shape[0] // P
    return pl.pallas_call(
        _knn_body,
        grid=(nb,),
        in_specs=[pl.BlockSpec((P, din), lambda b: (b, 0))],
        out_specs=pl.BlockSpec((1, K, P), lambda b: (b, 0, 0)),
        out_shape=jax.ShapeDtypeStruct((nb, K, P), jnp.int32),
        scratch_shapes=[pltpu.VMEM((P, P), jnp.float32)],
    )(x)


def _sc_gather(table, idx):
    e_total = idx.shape[0]
    d = table.shape[1]
    info = plsc.get_sparse_core_info()
    nw = info.num_cores * info.num_subcores
    e_per_w = e_total // nw
    ch = 128
    n_ch = e_per_w // ch
    mesh = plsc.VectorSubcoreMesh(core_axis_name="c", subcore_axis_name="s")

    @functools.partial(
        pl.kernel,
        mesh=mesh,
        out_type=jax.ShapeDtypeStruct((e_total, d), table.dtype),
        scratch_types=[
            pltpu.VMEM((e_per_w,), jnp.int32),
            pltpu.VMEM((2, ch, d), table.dtype),
            pltpu.SemaphoreType.DMA,
            pltpu.SemaphoreType.DMA,
        ],
    )
    def gather_k(table_hbm, idx_hbm, out_hbm, idx_v, rows_v, sem0, sem1):
        wid = lax.axis_index("s") * info.num_cores + lax.axis_index("c")
        base = wid * e_per_w
        pltpu.sync_copy(idx_hbm.at[pl.ds(base, e_per_w)], idx_v)
        sems = (sem0, sem1)

        def fire(c, buf):
            pltpu.async_copy(
                table_hbm.at[idx_v.at[pl.ds(c * ch, ch)]],
                rows_v.at[buf], sems[buf])

        def drain_and_store(c, buf):
            pltpu.make_async_copy(
                table_hbm.at[idx_v.at[pl.ds(c * ch, ch)]],
                rows_v.at[buf], sems[buf]).wait()
            pltpu.sync_copy(rows_v.at[buf],
                            out_hbm.at[pl.ds(base + c * ch, ch)])

        fire(0, 0)

        def loop_body(c, _):
            @pl.when(c + 1 < n_ch)
            def _fire_next():
                @pl.when(lax.rem(c + 1, 2) == 0)
                def _():
                    fire(c + 1, 0)

                @pl.when(lax.rem(c + 1, 2) == 1)
                def _():
                    fire(c + 1, 1)

            @pl.when(lax.rem(c, 2) == 0)
            def _w0():
                drain_and_store(c, 0)

            @pl.when(lax.rem(c, 2) == 1)
            def _w1():
                drain_and_store(c, 1)
            return 0

        lax.fori_loop(0, n_ch, loop_body, 0)

    return gather_k(table, idx)


def _edge_body(x_ref, g_ref, w0_ref, b0_ref, w1_ref, b1_ref, w2_ref, b2_ref,
               out_ref, *, c, din):
    i = pl.program_id(1)
    f2 = w2_ref.shape[1]
    xi = x_ref[pl.ds(i * c, c), :]
    xj = g_ref[0][:, :, :din]
    xib = jnp.broadcast_to(xi[None, :, :], (K, c, din))
    ein = jnp.concatenate([xib, xj - xib], axis=2).reshape(K * c, 2 * din)
    h = jnp.maximum(_bdot(ein, w0_ref[...]) + b0_ref[...], 0.0)
    h = jnp.maximum(_bdot(h, w1_ref[...]) + b1_ref[...], 0.0)
    h = jnp.maximum(_bdot(h, w2_ref[...]) + b2_ref[...], 0.0)
    out_ref[...] = jnp.max(h.reshape(K, c, f2), axis=0)


def _edge_call(x, g4, w0, b0, w1, b1, w2, b2):
    din = x.shape[1]
    gw = g4.shape[3]
    f1 = w0.shape[1]
    f2 = w2.shape[1]
    c = 128
    nc = P // c
    nb = x.shape[0] // P
    return pl.pallas_call(
        functools.partial(_edge_body, c=c, din=din),
        grid=(nb, nc),
        in_specs=[
            pl.BlockSpec((P, din), lambda b, i: (b, 0)),
            pl.BlockSpec((1, K, c, gw), lambda b, i: (b, 0, i, 0)),
            pl.BlockSpec((2 * din, f1), lambda b, i: (0, 0)),
            pl.BlockSpec((1, f1), lambda b, i: (0, 0)),
            pl.BlockSpec((f1, f1), lambda b, i: (0, 0)),
            pl.BlockSpec((1, f1), lambda b, i: (0, 0)),
            pl.BlockSpec((f1, f2), lambda b, i: (0, 0)),
            pl.BlockSpec((1, f2), lambda b, i: (0, 0)),
        ],
        out_specs=pl.BlockSpec((c, f2), lambda b, i: (b * nc + i, 0)),
        out_shape=jax.ShapeDtypeStruct((x.shape[0], f2), jnp.float32),
    )(x, g4, w0, b0.reshape(1, f1), w1, b1.reshape(1, f1),
      w2, b2.reshape(1, f2))


def _head_body(x1_ref, x2_ref, w0_ref, b0_ref, wl1a_ref, wl1b_ref, wl1c_ref,
               bl1_ref, wl2_ref, bl2_ref, out_ref):
    x1 = x1_ref[...]
    x2 = x2_ref[...]
    h = jnp.maximum(_bdot(x2, w0_ref[...]) + b0_ref[...], 0.0)
    pooled = jnp.max(h, axis=0, keepdims=True)
    z = (_bdot(x1, wl1a_ref[...]) + _bdot(x2, wl1b_ref[...])
         + _bdot(pooled, wl1c_ref[...]) + bl1_ref[...])
    z = jnp.maximum(z, 0.0)
    out_ref[...] = jnp.maximum(_bdot(z, wl2_ref[...]) + bl2_ref[...], 0.0)


def _head_call(x1, x2, w0, b0, wl1, bl1, wl2, bl2):
    nb = x1.shape[0] // P
    return pl.pallas_call(
        _head_body,
        grid=(nb,),
        in_specs=[
            pl.BlockSpec((P, 64), lambda b: (b, 0)),
            pl.BlockSpec((P, 256), lambda b: (b, 0)),
            pl.BlockSpec((256, 512), lambda b: (0, 0)),
            pl.BlockSpec((1, 512), lambda b: (0, 0)),
            pl.BlockSpec((64, 256), lambda b: (0, 0)),
            pl.BlockSpec((256, 256), lambda b: (0, 0)),
            pl.BlockSpec((512, 256), lambda b: (0, 0)),
            pl.BlockSpec((1, 256), lambda b: (0, 0)),
            pl.BlockSpec((256, 256), lambda b: (0, 0)),
            pl.BlockSpec((1, 256), lambda b: (0, 0)),
        ],
        out_specs=pl.BlockSpec((P, 256), lambda b: (b, 0)),
        out_shape=jax.ShapeDtypeStruct((x1.shape[0], 256), jnp.float32),
    )(x1, x2, w0, b0.reshape(1, 512), wl1[:64], wl1[64:320], wl1[320:],
      bl1.reshape(1, 256), wl2, bl2.reshape(1, 256))


def _pad128(x):
    return jnp.concatenate(
        [x, jnp.zeros((x.shape[0], 128 - x.shape[1]), x.dtype)], axis=1)


def kernel(pos, t, batch, W1_0, b1_0, W1_1, b1_1, W1_2, b1_2,
           W2_0, b2_0, W2_1, b2_1, W2_2, b2_2, W0, b0, Wl1, bl1, Wl2, bl2):
    del t, batch
    groups = 4
    bg = B // groups
    outs = []
    for g in range(groups):
        sl = slice(g * bg * P, (g + 1) * bg * P)
        pos_g = pos[sl]
        idx1 = _knn_call(pos_g)
        g1 = _sc_gather(_pad128(pos_g), idx1.reshape(-1))
        x1 = _edge_call(pos_g, g1.reshape(bg, K, P, 128), W1_0, b1_0,
                        W1_1, b1_1, W1_2, b1_2)
        idx2 = _knn_call(x1)
        g2 = _sc_gather(_pad128(x1), idx2.reshape(-1))
        x2 = _edge_call(x1, g2.reshape(bg, K, P, 128), W2_0, b2_0,
                        W2_1, b2_1, W2_2, b2_2)
        outs.append(_head_call(x1, x2, W0, b0, Wl1, bl1, Wl2, bl2))
    return jnp.concatenate(outs, axis=0)


# speedup vs baseline: 10.8963x; 1.1338x over previous
import functools

import jax
import jax.numpy as jnp
from jax import lax
from jax.experimental import pallas as pl
from jax.experimental.pallas import tpu as pltpu
from jax.experimental.pallas import tpu_sc as plsc

B = 16
P = 512
N = B * P
K = 30


def _bdot(a, b):
    return lax.dot_general(a.astype(jnp.bfloat16), b.astype(jnp.bfloat16),
                           (((1,), (0,)), ((), ())),
                           preferred_element_type=jnp.float32)


def _knn_body(x_ref, idx_ref, d_ref):
    boff = pl.program_id(0) * P
    x = x_ref[...]
    xh = x.astype(jnp.bfloat16)
    prod = lax.dot_general(xh, xh, (((1,), (1,)), ((), ())),
                           preferred_element_type=jnp.float32)
    sq = jnp.sum(x * x, axis=1, keepdims=True)
    iota0 = lax.broadcasted_iota(jnp.int32, (P, P), 0)
    iota1 = lax.broadcasted_iota(jnp.int32, (P, P), 1)
    d_ref[...] = jnp.where(iota0 == iota1, jnp.float32(1e10), sq - 2.0 * prod)

    for k in range(K):
        d = d_ref[...]
        colmin = jnp.min(d, axis=0, keepdims=True)
        cand = jnp.where(d == colmin, iota0, P)
        sel = jnp.min(cand, axis=0, keepdims=True)
        idx_ref[0, pl.ds(k, 1), :] = sel + boff
        d_ref[...] = jnp.where(iota0 == sel, jnp.float32(2e10), d)


def _knn_call(x):
    din = x.shape[1]
    nb = x.shape[0] // P
    return pl.pallas_call(
        _knn_body,
        grid=(nb,),
        in_specs=[pl.BlockSpec((P, din), lambda b: (b, 0))],
        out_specs=pl.BlockSpec((1, K, P), lambda b: (b, 0, 0)),
        out_shape=jax.ShapeDtypeStruct((nb, K, P), jnp.int32),
        scratch_shapes=[pltpu.VMEM((P, P), jnp.float32)],
    )(x)


def _sc_gather(table, idx):
    e_total = idx.shape[0]
    d = table.shape[1]
    info = plsc.get_sparse_core_info()
    nw = info.num_cores * info.num_subcores
    e_per_w = e_total // nw
    ch = 128
    n_ch = e_per_w // ch
    nbuf = 3
    mesh = plsc.VectorSubcoreMesh(core_axis_name="c", subcore_axis_name="s")

    @functools.partial(
        pl.kernel,
        mesh=mesh,
        out_type=jax.ShapeDtypeStruct((e_total, d), table.dtype),
        scratch_types=[
            pltpu.VMEM((e_per_w,), jnp.int32),
            pltpu.VMEM((nbuf, ch, d), table.dtype),
            pltpu.SemaphoreType.DMA,
            pltpu.SemaphoreType.DMA,
            pltpu.SemaphoreType.DMA,
        ],
    )
    def gather_k(table_hbm, idx_hbm, out_hbm, idx_v, rows_v, sem0, sem1, sem2):
        wid = lax.axis_index("s") * info.num_cores + lax.axis_index("c")
        base = wid * e_per_w
        pltpu.sync_copy(idx_hbm.at[pl.ds(base, e_per_w)], idx_v)
        sems = (sem0, sem1, sem2)

        def fire(c, buf):
            pltpu.async_copy(
                table_hbm.at[idx_v.at[pl.ds(c * ch, ch)]],
                rows_v.at[buf], sems[buf])

        def drain_and_store(c, buf):
            pltpu.make_async_copy(
                table_hbm.at[idx_v.at[pl.ds(c * ch, ch)]],
                rows_v.at[buf], sems[buf]).wait()
            pltpu.sync_copy(rows_v.at[buf],
                            out_hbm.at[pl.ds(base + c * ch, ch)])

        fire(0, 0)
        fire(1, 1)

        def loop_body(c, _):
            @pl.when(c + 2 < n_ch)
            def _fire_next():
                for b in range(nbuf):
                    @pl.when(lax.rem(c + 2, nbuf) == b)
                    def _(b=b):
                        fire(c + 2, b)

            for b in range(nbuf):
                @pl.when(lax.rem(c, nbuf) == b)
                def _(b=b):
                    drain_and_store(c, b)
            return 0

        lax.fori_loop(0, n_ch, loop_body, 0)

    return gather_k(table, idx)


def _edge_body(x_ref, g_ref, w0t_ref, w0b_ref, b0_ref, w1_ref, b1_ref,
               w2_ref, b2_ref, out_ref, *, c, din):
    i = pl.program_id(1)
    f1 = w1_ref.shape[0]
    f2 = w2_ref.shape[1]
    xi = x_ref[pl.ds(i * c, c), :]
    t1 = _bdot(xi, w0t_ref[...]) + b0_ref[...]
    xj = g_ref[0][:, :, :din]
    dx = (xj - xi[None, :, :]).reshape(K * c, din)
    t2 = _bdot(dx, w0b_ref[...])
    h = jnp.maximum(t2.reshape(K, c, f1) + t1[None, :, :], 0.0)
    h = h.reshape(K * c, f1)
    h = jnp.maximum(_bdot(h, w1_ref[...]) + b1_ref[...], 0.0)
    h = jnp.maximum(_bdot(h, w2_ref[...]) + b2_ref[...], 0.0)
    out_ref[...] = jnp.max(h.reshape(K, c, f2), axis=0)


def _edge_call(x, g4, w0, b0, w1, b1, w2, b2):
    din = x.shape[1]
    gw = g4.shape[3]
    f1 = w0.shape[1]
    f2 = w2.shape[1]
    c = 256
    nc = P // c
    nb = x.shape[0] // P
    return pl.pallas_call(
        functools.partial(_edge_body, c=c, din=din),
        grid=(nb, nc),
        in_specs=[
            pl.BlockSpec((P, din), lambda b, i: (b, 0)),
            pl.BlockSpec((1, K, c, gw), lambda b, i: (b, 0, i, 0)),
            pl.BlockSpec((din, f1), lambda b, i: (0, 0)),
            pl.BlockSpec((din, f1), lambda b, i: (0, 0)),
            pl.BlockSpec((1, f1), lambda b, i: (0, 0)),
            pl.BlockSpec((f1, f1), lambda b, i: (0, 0)),
            pl.BlockSpec((1, f1), lambda b, i: (0, 0)),
            pl.BlockSpec((f1, f2), lambda b, i: (0, 0)),
            pl.BlockSpec((1, f2), lambda b, i: (0, 0)),
        ],
        out_specs=pl.BlockSpec((c, f2), lambda b, i: (b * nc + i, 0)),
        out_shape=jax.ShapeDtypeStruct((x.shape[0], f2), jnp.float32),
    )(x, g4, w0[:din], w0[din:], b0.reshape(1, f1), w1, b1.reshape(1, f1),
      w2, b2.reshape(1, f2))


def _head_body(x1_ref, x2_ref, w0_ref, b0_ref, wl1a_ref, wl1b_ref, wl1c_ref,
               bl1_ref, wl2_ref, bl2_ref, out_ref):
    x1 = x1_ref[...]
    x2 = x2_ref[...]
    h = jnp.maximum(_bdot(x2, w0_ref[...]) + b0_ref[...], 0.0)
    pooled = jnp.max(h, axis=0, keepdims=True)
    z = (_bdot(x1, wl1a_ref[...]) + _bdot(x2, wl1b_ref[...])
         + _bdot(pooled, wl1c_ref[...]) + bl1_ref[...])
    z = jnp.maximum(z, 0.0)
    out_ref[...] = jnp.maximum(_bdot(z, wl2_ref[...]) + bl2_ref[...], 0.0)


def _head_call(x1, x2, w0, b0, wl1, bl1, wl2, bl2):
    nb = x1.shape[0] // P
    return pl.pallas_call(
        _head_body,
        grid=(nb,),
        in_specs=[
            pl.BlockSpec((P, 64), lambda b: (b, 0)),
            pl.BlockSpec((P, 256), lambda b: (b, 0)),
            pl.BlockSpec((256, 512), lambda b: (0, 0)),
            pl.BlockSpec((1, 512), lambda b: (0, 0)),
            pl.BlockSpec((64, 256), lambda b: (0, 0)),
            pl.BlockSpec((256, 256), lambda b: (0, 0)),
            pl.BlockSpec((512, 256), lambda b: (0, 0)),
            pl.BlockSpec((1, 256), lambda b: (0, 0)),
            pl.BlockSpec((256, 256), lambda b: (0, 0)),
            pl.BlockSpec((1, 256), lambda b: (0, 0)),
        ],
        out_specs=pl.BlockSpec((P, 256), lambda b: (b, 0)),
        out_shape=jax.ShapeDtypeStruct((x1.shape[0], 256), jnp.float32),
    )(x1, x2, w0, b0.reshape(1, 512), wl1[:64], wl1[64:320], wl1[320:],
      bl1.reshape(1, 256), wl2, bl2.reshape(1, 256))


def _pad128(x):
    return jnp.concatenate(
        [x, jnp.zeros((x.shape[0], 128 - x.shape[1]), x.dtype)], axis=1)


def kernel(pos, t, batch, W1_0, b1_0, W1_1, b1_1, W1_2, b1_2,
           W2_0, b2_0, W2_1, b2_1, W2_2, b2_2, W0, b0, Wl1, bl1, Wl2, bl2):
    del t, batch
    groups = 4
    bg = B // groups
    outs = []
    for g in range(groups):
        sl = slice(g * bg * P, (g + 1) * bg * P)
        pos_g = pos[sl]
        idx1 = _knn_call(pos_g)
        g1 = _sc_gather(_pad128(pos_g), idx1.reshape(-1))
        x1 = _edge_call(pos_g, g1.reshape(bg, K, P, 128), W1_0, b1_0,
                        W1_1, b1_1, W1_2, b1_2)
        idx2 = _knn_call(x1)
        g2 = _sc_gather(_pad128(x1), idx2.reshape(-1))
        x2 = _edge_call(x1, g2.reshape(bg, K, P, 128), W2_0, b2_0,
                        W2_1, b2_1, W2_2, b2_2)
        outs.append(_head_call(x1, x2, W0, b0, Wl1, bl1, Wl2, bl2))
    return jnp.concatenate(outs, axis=0)


# speedup vs baseline: 10.9251x; 1.0026x over previous
import functools

import jax
import jax.numpy as jnp
from jax import lax
from jax.experimental import pallas as pl
from jax.experimental.pallas import tpu as pltpu
from jax.experimental.pallas import tpu_sc as plsc

B = 16
P = 512
N = B * P
K = 30


def _bdot(a, b):
    return lax.dot_general(a.astype(jnp.bfloat16), b.astype(jnp.bfloat16),
                           (((1,), (0,)), ((), ())),
                           preferred_element_type=jnp.float32)


def _knn_body(x_ref, idx_ref, d_ref):
    boff = pl.program_id(0) * P
    x = x_ref[...]
    xh = x.astype(jnp.bfloat16)
    prod = lax.dot_general(xh, xh, (((1,), (1,)), ((), ())),
                           preferred_element_type=jnp.float32)
    sq = jnp.sum(x * x, axis=1, keepdims=True)
    iota0 = lax.broadcasted_iota(jnp.int32, (P, P), 0)
    iota1 = lax.broadcasted_iota(jnp.int32, (P, P), 1)
    d_ref[...] = jnp.where(iota0 == iota1, jnp.float32(1e10), sq - 2.0 * prod)

    for k in range(K):
        d = d_ref[...]
        colmin = jnp.min(d, axis=0, keepdims=True)
        cand = jnp.where(d == colmin, iota0, P)
        sel = jnp.min(cand, axis=0, keepdims=True)
        idx_ref[0, pl.ds(k, 1), :] = sel + boff
        d_ref[...] = jnp.where(iota0 == sel, jnp.float32(2e10), d)


def _knn_call(x):
    din = x.shape[1]
    nb = x.shape[0] // P
    return pl.pallas_call(
        _knn_body,
        grid=(nb,),
        in_specs=[pl.BlockSpec((P, din), lambda b: (b, 0))],
        out_specs=pl.BlockSpec((1, K, P), lambda b: (b, 0, 0)),
        out_shape=jax.ShapeDtypeStruct((nb, K, P), jnp.int32),
        scratch_shapes=[pltpu.VMEM((P, P), jnp.float32)],
    )(x)


def _sc_gather(table, idx):
    e_total = idx.shape[0]
    d = table.shape[1]
    info = plsc.get_sparse_core_info()
    nw = info.num_cores * info.num_subcores
    e_per_w = e_total // nw
    ch = 128
    n_ch = e_per_w // ch
    nbuf = 3
    n_rounds = n_ch // nbuf
    assert n_ch % nbuf == 0
    mesh = plsc.VectorSubcoreMesh(core_axis_name="c", subcore_axis_name="s")

    @functools.partial(
        pl.kernel,
        mesh=mesh,
        out_type=jax.ShapeDtypeStruct((e_total, d), table.dtype),
        scratch_types=[
            pltpu.VMEM((e_per_w,), jnp.int32),
            pltpu.VMEM((2, nbuf * ch, d), table.dtype),
            pltpu.SemaphoreType.DMA,
            pltpu.SemaphoreType.DMA,
            pltpu.SemaphoreType.DMA,
            pltpu.SemaphoreType.DMA,
        ],
    )
    def gather_k(table_hbm, idx_hbm, out_hbm, idx_v, rows_v,
                 gsem0, gsem1, ssem0, ssem1):
        wid = lax.axis_index("s") * info.num_cores + lax.axis_index("c")
        base = wid * e_per_w
        pltpu.sync_copy(idx_hbm.at[pl.ds(base, e_per_w)], idx_v)
        gsems = (gsem0, gsem1)
        ssems = (ssem0, ssem1)

        def fire_round(r, s):
            for b in range(nbuf):
                pltpu.async_copy(
                    table_hbm.at[idx_v.at[pl.ds((r * nbuf + b) * ch, ch)]],
                    rows_v.at[s, pl.ds(b * ch, ch)], gsems[s])

        def drain_round(r, s):
            for b in range(nbuf):
                pltpu.make_async_copy(
                    table_hbm.at[idx_v.at[pl.ds((r * nbuf + b) * ch, ch)]],
                    rows_v.at[s, pl.ds(b * ch, ch)], gsems[s]).wait()
            pltpu.async_copy(rows_v.at[s],
                             out_hbm.at[pl.ds(base + r * nbuf * ch, nbuf * ch)],
                             ssems[s])

        def wait_store(s):
            pltpu.make_async_copy(
                rows_v.at[s],
                out_hbm.at[pl.ds(base, nbuf * ch)], ssems[s]).wait()

        fire_round(0, 0)

        def round_body(r, _):
            for s in range(2):
                @pl.when(lax.rem(r, 2) == s)
                def _(s=s):
                    @pl.when(r + 1 < n_rounds)
                    def _fire():
                        @pl.when(r >= 1)
                        def _ws():
                            wait_store(1 - s)
                        fire_round(r + 1, 1 - s)
                    drain_round(r, s)
            return 0

        lax.fori_loop(0, n_rounds, round_body, 0)
        @pl.when(n_rounds >= 2)
        def _():
            for s in range(2):
                @pl.when(lax.rem(n_rounds - 2, 2) == s)
                def _(s=s):
                    wait_store(s)
        for s in range(2):
            @pl.when(lax.rem(n_rounds - 1, 2) == s)
            def _(s=s):
                wait_store(s)

    return gather_k(table, idx)


def _edge_body(x_ref, g_ref, w0t_ref, w0b_ref, b0_ref, w1_ref, b1_ref,
               w2_ref, b2_ref, out_ref, *, c, din):
    i = pl.program_id(1)
    f1 = w1_ref.shape[0]
    f2 = w2_ref.shape[1]
    xi = x_ref[pl.ds(i * c, c), :]
    t1 = _bdot(xi, w0t_ref[...]) + b0_ref[...]
    xj = g_ref[0][:, :, :din]
    dx = (xj - xi[None, :, :]).reshape(K * c, din)
    t2 = _bdot(dx, w0b_ref[...])
    h = jnp.maximum(t2.reshape(K, c, f1) + t1[None, :, :], 0.0)
    h = h.reshape(K * c, f1)
    h = jnp.maximum(_bdot(h, w1_ref[...]) + b1_ref[...], 0.0)
    h = jnp.maximum(_bdot(h, w2_ref[...]) + b2_ref[...], 0.0)
    out_ref[...] = jnp.max(h.reshape(K, c, f2), axis=0)


def _edge_call(x, g4, w0, b0, w1, b1, w2, b2):
    din = x.shape[1]
    gw = g4.shape[3]
    f1 = w0.shape[1]
    f2 = w2.shape[1]
    c = 256
    nc = P // c
    nb = x.shape[0] // P
    return pl.pallas_call(
        functools.partial(_edge_body, c=c, din=din),
        grid=(nb, nc),
        in_specs=[
            pl.BlockSpec((P, din), lambda b, i: (b, 0)),
            pl.BlockSpec((1, K, c, gw), lambda b, i: (b, 0, i, 0)),
            pl.BlockSpec((din, f1), lambda b, i: (0, 0)),
            pl.BlockSpec((din, f1), lambda b, i: (0, 0)),
            pl.BlockSpec((1, f1), lambda b, i: (0, 0)),
            pl.BlockSpec((f1, f1), lambda b, i: (0, 0)),
            pl.BlockSpec((1, f1), lambda b, i: (0, 0)),
            pl.BlockSpec((f1, f2), lambda b, i: (0, 0)),
            pl.BlockSpec((1, f2), lambda b, i: (0, 0)),
        ],
        out_specs=pl.BlockSpec((c, f2), lambda b, i: (b * nc + i, 0)),
        out_shape=jax.ShapeDtypeStruct((x.shape[0], f2), jnp.float32),
    )(x, g4, w0[:din], w0[din:], b0.reshape(1, f1), w1, b1.reshape(1, f1),
      w2, b2.reshape(1, f2))


def _head_body(x1_ref, x2_ref, w0_ref, b0_ref, wl1a_ref, wl1b_ref, wl1c_ref,
               bl1_ref, wl2_ref, bl2_ref, out_ref):
    x1 = x1_ref[...]
    x2 = x2_ref[...]
    h = jnp.maximum(_bdot(x2, w0_ref[...]) + b0_ref[...], 0.0)
    pooled = jnp.max(h, axis=0, keepdims=True)
    z = (_bdot(x1, wl1a_ref[...]) + _bdot(x2, wl1b_ref[...])
         + _bdot(pooled, wl1c_ref[...]) + bl1_ref[...])
    z = jnp.maximum(z, 0.0)
    out_ref[...] = jnp.maximum(_bdot(z, wl2_ref[...]) + bl2_ref[...], 0.0)


def _head_call(x1, x2, w0, b0, wl1, bl1, wl2, bl2):
    nb = x1.shape[0] // P
    return pl.pallas_call(
        _head_body,
        grid=(nb,),
        in_specs=[
            pl.BlockSpec((P, 64), lambda b: (b, 0)),
            pl.BlockSpec((P, 256), lambda b: (b, 0)),
            pl.BlockSpec((256, 512), lambda b: (0, 0)),
            pl.BlockSpec((1, 512), lambda b: (0, 0)),
            pl.BlockSpec((64, 256), lambda b: (0, 0)),
            pl.BlockSpec((256, 256), lambda b: (0, 0)),
            pl.BlockSpec((512, 256), lambda b: (0, 0)),
            pl.BlockSpec((1, 256), lambda b: (0, 0)),
            pl.BlockSpec((256, 256), lambda b: (0, 0)),
            pl.BlockSpec((1, 256), lambda b: (0, 0)),
        ],
        out_specs=pl.BlockSpec((P, 256), lambda b: (b, 0)),
        out_shape=jax.ShapeDtypeStruct((x1.shape[0], 256), jnp.float32),
    )(x1, x2, w0, b0.reshape(1, 512), wl1[:64], wl1[64:320], wl1[320:],
      bl1.reshape(1, 256), wl2, bl2.reshape(1, 256))


def _pad128(x):
    return jnp.concatenate(
        [x, jnp.zeros((x.shape[0], 128 - x.shape[1]), x.dtype)], axis=1)


def kernel(pos, t, batch, W1_0, b1_0, W1_1, b1_1, W1_2, b1_2,
           W2_0, b2_0, W2_1, b2_1, W2_2, b2_2, W0, b0, Wl1, bl1, Wl2, bl2):
    del t, batch
    groups = 4
    bg = B // groups
    pos_gs = [pos[g * bg * P:(g + 1) * bg * P] for g in range(groups)]
    idx1s = [_knn_call(p) for p in pos_gs]
    g1s = [_sc_gather(_pad128(p), i.reshape(-1))
           for p, i in zip(pos_gs, idx1s)]
    x1s = [_edge_call(p, g.reshape(bg, K, P, 128), W1_0, b1_0,
                      W1_1, b1_1, W1_2, b1_2)
           for p, g in zip(pos_gs, g1s)]
    idx2s = [_knn_call(x) for x in x1s]
    g2s = [_sc_gather(_pad128(x), i.reshape(-1))
           for x, i in zip(x1s, idx2s)]
    x2s = [_edge_call(x, g.reshape(bg, K, P, 128), W2_0, b2_0,
                      W2_1, b2_1, W2_2, b2_2)
           for x, g in zip(x1s, g2s)]
    outs = [_head_call(x1, x2, W0, b0, Wl1, bl1, Wl2, bl2)
            for x1, x2 in zip(x1s, x2s)]
    return jnp.concatenate(outs, axis=0)


# speedup vs baseline: 10.9316x; 1.0006x over previous
import functools

import jax
import jax.numpy as jnp
from jax import lax
from jax.experimental import pallas as pl
from jax.experimental.pallas import tpu as pltpu
from jax.experimental.pallas import tpu_sc as plsc

B = 16
P = 512
N = B * P
K = 30


def _bdot(a, b):
    return lax.dot_general(a.astype(jnp.bfloat16), b.astype(jnp.bfloat16),
                           (((1,), (0,)), ((), ())),
                           preferred_element_type=jnp.float32)


def _knn_body(x_ref, idx_ref, d_ref):
    boff = pl.program_id(0) * P
    x = x_ref[...]
    xh = x.astype(jnp.bfloat16)
    prod = lax.dot_general(xh, xh, (((1,), (1,)), ((), ())),
                           preferred_element_type=jnp.float32)
    sq = jnp.sum(x * x, axis=1, keepdims=True)
    iota0 = lax.broadcasted_iota(jnp.int32, (P, P), 0)
    iota1 = lax.broadcasted_iota(jnp.int32, (P, P), 1)
    d_ref[...] = jnp.where(iota0 == iota1, jnp.float32(1e10), sq - 2.0 * prod)

    for k in range(K):
        d = d_ref[...]
        colmin = jnp.min(d, axis=0, keepdims=True)
        cand = jnp.where(d == colmin, iota0, P)
        sel = jnp.min(cand, axis=0, keepdims=True)
        idx_ref[0, pl.ds(k, 1), :] = sel + boff
        d_ref[...] = jnp.where(iota0 == sel, jnp.float32(2e10), d)


def _knn_call(x):
    din = x.shape[1]
    nb = x.shape[0] // P
    return pl.pallas_call(
        _knn_body,
        grid=(nb,),
        in_specs=[pl.BlockSpec((P, din), lambda b: (b, 0))],
        out_specs=pl.BlockSpec((1, K, P), lambda b: (b, 0, 0)),
        out_shape=jax.ShapeDtypeStruct((nb, K, P), jnp.int32),
        scratch_shapes=[pltpu.VMEM((P, P), jnp.float32)],
    )(x)


def _sc_gather(table, idx):
    e_total = idx.shape[0]
    d = table.shape[1]
    info = plsc.get_sparse_core_info()
    nw = info.num_cores * info.num_subcores
    e_per_w = e_total // nw
    ch = 128
    n_ch = e_per_w // ch
    nbuf = 3
    n_rounds = n_ch // nbuf
    assert n_ch % nbuf == 0
    mesh = plsc.VectorSubcoreMesh(core_axis_name="c", subcore_axis_name="s")

    @functools.partial(
        pl.kernel,
        mesh=mesh,
        out_type=jax.ShapeDtypeStruct((e_total, d), table.dtype),
        scratch_types=[
            pltpu.VMEM((e_per_w,), jnp.int32),
            pltpu.VMEM((2, nbuf * ch, d), table.dtype),
            pltpu.SemaphoreType.DMA,
            pltpu.SemaphoreType.DMA,
            pltpu.SemaphoreType.DMA,
            pltpu.SemaphoreType.DMA,
        ],
    )
    def gather_k(table_hbm, idx_hbm, out_hbm, idx_v, rows_v,
                 gsem0, gsem1, ssem0, ssem1):
        wid = lax.axis_index("s") * info.num_cores + lax.axis_index("c")
        base = wid * e_per_w
        pltpu.sync_copy(idx_hbm.at[pl.ds(base, e_per_w)], idx_v)
        gsems = (gsem0, gsem1)
        ssems = (ssem0, ssem1)

        def fire_round(r, s):
            for b in range(nbuf):
                pltpu.async_copy(
                    table_hbm.at[idx_v.at[pl.ds((r * nbuf + b) * ch, ch)]],
                    rows_v.at[s, pl.ds(b * ch, ch)], gsems[s])

        def drain_round(r, s):
            for b in range(nbuf):
                pltpu.make_async_copy(
                    table_hbm.at[idx_v.at[pl.ds((r * nbuf + b) * ch, ch)]],
                    rows_v.at[s, pl.ds(b * ch, ch)], gsems[s]).wait()
            pltpu.async_copy(rows_v.at[s],
                             out_hbm.at[pl.ds(base + r * nbuf * ch, nbuf * ch)],
                             ssems[s])

        def wait_store(s):
            pltpu.make_async_copy(
                rows_v.at[s],
                out_hbm.at[pl.ds(base, nbuf * ch)], ssems[s]).wait()

        fire_round(0, 0)

        def round_body(r, _):
            for s in range(2):
                @pl.when(lax.rem(r, 2) == s)
                def _(s=s):
                    @pl.when(r + 1 < n_rounds)
                    def _fire():
                        @pl.when(r >= 1)
                        def _ws():
                            wait_store(1 - s)
                        fire_round(r + 1, 1 - s)
                    drain_round(r, s)
            return 0

        lax.fori_loop(0, n_rounds, round_body, 0)
        @pl.when(n_rounds >= 2)
        def _():
            for s in range(2):
                @pl.when(lax.rem(n_rounds - 2, 2) == s)
                def _(s=s):
                    wait_store(s)
        for s in range(2):
            @pl.when(lax.rem(n_rounds - 1, 2) == s)
            def _(s=s):
                wait_store(s)

    return gather_k(table, idx)


def _edge_tail(xi, xj, w0t_ref, w0b_ref, b0_ref, w1_ref, b1_ref,
               w2_ref, b2_ref, out_ref, c, din):
    f1 = w1_ref.shape[0]
    f2 = w2_ref.shape[1]
    t1 = _bdot(xi, w0t_ref[...]) + b0_ref[...]
    dx = (xj - xi[None, :, :]).reshape(K * c, din)
    t2 = _bdot(dx, w0b_ref[...])
    h = jnp.maximum(t2.reshape(K, c, f1) + t1[None, :, :], 0.0)
    h = h.reshape(K * c, f1)
    h = jnp.maximum(_bdot(h, w1_ref[...]) + b1_ref[...], 0.0)
    h = jnp.maximum(_bdot(h, w2_ref[...]) + b2_ref[...], 0.0)
    out_ref[...] = jnp.max(h.reshape(K, c, f2), axis=0)


def _edge_body(x_ref, g_ref, w0t_ref, w0b_ref, b0_ref, w1_ref, b1_ref,
               w2_ref, b2_ref, out_ref, *, c, din):
    i = pl.program_id(1)
    xi = x_ref[pl.ds(i * c, c), :]
    xj = g_ref[0][:, :, :din]
    _edge_tail(xi, xj, w0t_ref, w0b_ref, b0_ref, w1_ref, b1_ref,
               w2_ref, b2_ref, out_ref, c, din)


def _edge_call(x, g4, w0, b0, w1, b1, w2, b2):
    din = x.shape[1]
    gw = g4.shape[3]
    f1 = w0.shape[1]
    f2 = w2.shape[1]
    c = 256
    nc = P // c
    nb = x.shape[0] // P
    return pl.pallas_call(
        functools.partial(_edge_body, c=c, din=din),
        grid=(nb, nc),
        in_specs=[
            pl.BlockSpec((P, din), lambda b, i: (b, 0)),
            pl.BlockSpec((1, K, c, gw), lambda b, i: (b, 0, i, 0)),
            pl.BlockSpec((din, f1), lambda b, i: (0, 0)),
            pl.BlockSpec((din, f1), lambda b, i: (0, 0)),
            pl.BlockSpec((1, f1), lambda b, i: (0, 0)),
            pl.BlockSpec((f1, f1), lambda b, i: (0, 0)),
            pl.BlockSpec((1, f1), lambda b, i: (0, 0)),
            pl.BlockSpec((f1, f2), lambda b, i: (0, 0)),
            pl.BlockSpec((1, f2), lambda b, i: (0, 0)),
        ],
        out_specs=pl.BlockSpec((c, f2), lambda b, i: (b * nc + i, 0)),
        out_shape=jax.ShapeDtypeStruct((x.shape[0], f2), jnp.float32),
    )(x, g4, w0[:din], w0[din:], b0.reshape(1, f1), w1, b1.reshape(1, f1),
      w2, b2.reshape(1, f2))


def _head_body(x1_ref, x2_ref, w0_ref, b0_ref, wl1a_ref, wl1b_ref, wl1c_ref,
               bl1_ref, wl2_ref, bl2_ref, out_ref):
    x1 = x1_ref[...]
    x2 = x2_ref[...]
    h = jnp.maximum(_bdot(x2, w0_ref[...]) + b0_ref[...], 0.0)
    pooled = jnp.max(h, axis=0, keepdims=True)
    z = (_bdot(x1, wl1a_ref[...]) + _bdot(x2, wl1b_ref[...])
         + _bdot(pooled, wl1c_ref[...]) + bl1_ref[...])
    z = jnp.maximum(z, 0.0)
    out_ref[...] = jnp.maximum(_bdot(z, wl2_ref[...]) + bl2_ref[...], 0.0)


def _head_call(x1, x2, w0, b0, wl1, bl1, wl2, bl2):
    nb = x1.shape[0] // P
    return pl.pallas_call(
        _head_body,
        grid=(nb,),
        in_specs=[
            pl.BlockSpec((P, 64), lambda b: (b, 0)),
            pl.BlockSpec((P, 256), lambda b: (b, 0)),
            pl.BlockSpec((256, 512), lambda b: (0, 0)),
            pl.BlockSpec((1, 512), lambda b: (0, 0)),
            pl.BlockSpec((64, 256), lambda b: (0, 0)),
            pl.BlockSpec((256, 256), lambda b: (0, 0)),
            pl.BlockSpec((512, 256), lambda b: (0, 0)),
            pl.BlockSpec((1, 256), lambda b: (0, 0)),
            pl.BlockSpec((256, 256), lambda b: (0, 0)),
            pl.BlockSpec((1, 256), lambda b: (0, 0)),
        ],
        out_specs=pl.BlockSpec((P, 256), lambda b: (b, 0)),
        out_shape=jax.ShapeDtypeStruct((x1.shape[0], 256), jnp.float32),
    )(x1, x2, w0, b0.reshape(1, 512), wl1[:64], wl1[64:320], wl1[320:],
      bl1.reshape(1, 256), wl2, bl2.reshape(1, 256))


def _pad128(x):
    return jnp.concatenate(
        [x, jnp.zeros((x.shape[0], 128 - x.shape[1]), x.dtype)], axis=1)


def kernel(pos, t, batch, W1_0, b1_0, W1_1, b1_1, W1_2, b1_2,
           W2_0, b2_0, W2_1, b2_1, W2_2, b2_2, W0, b0, Wl1, bl1, Wl2, bl2):
    del t, batch
    groups = 4
    bg = B // groups
    pos_gs = [pos[g * bg * P:(g + 1) * bg * P] for g in range(groups)]
    idx1s = [_knn_call(p) for p in pos_gs]
    g1s = [_sc_gather(_pad128(p), i.reshape(-1))
           for p, i in zip(pos_gs, idx1s)]
    x1s = [_edge_call(p, g.reshape(bg, K, P, 128), W1_0, b1_0,
                      W1_1, b1_1, W1_2, b1_2)
           for p, g in zip(pos_gs, g1s)]
    idx2s = [_knn_call(x) for x in x1s]
    g2s = [_sc_gather(_pad128(x), i.reshape(-1))
           for x, i in zip(x1s, idx2s)]
    x2s = [_edge_call(x, g.reshape(bg, K, P, 128), W2_0, b2_0,
                      W2_1, b2_1, W2_2, b2_2)
           for x, g in zip(x1s, g2s)]
    outs = [_head_call(x1, x2, W0, b0, Wl1, bl1, Wl2, bl2)
            for x1, x2 in zip(x1s, x2s)]
    return jnp.concatenate(outs, axis=0)
